# baseline (speedup 1.0000x reference)
import jax
import jax.numpy as jnp
from jax import lax
from jax.experimental import pallas as pl
from jax.experimental.pallas import tpu as pltpu
from jax.experimental.pallas import tpu_sc as plsc

B = 16384
S_DIM = 64
T_DIM = 32
R_DIM = S_DIM + T_DIM
CYCLE = 365

NC = 2
NS = 16
LANES = 16
NW = NC * NS
PER_W = B // NW
CHUNK = 64
NCHUNK = PER_W // CHUNK
NGROUP = CHUNK // LANES

_INV2PI = 0.15915494309189535
_MAGIC = 12582912.0
_C1 = 6.28125
_C2 = 1.9353071795864769e-3
_S0 = 0.9999782156662488
_S1 = -0.16662248279410358
_S2 = 0.008308176673817783
_S3 = -0.00019252550586158768
_S4 = 2.141589485971096e-06


def _psin(x):
    k = (x * _INV2PI + _MAGIC) - _MAGIC
    r = (x - k * _C1) - k * _C2
    t = r * r
    p = _S4 * t + _S3
    p = p * t + _S2
    p = p * t + _S1
    p = p * t + _S0
    return p * r


def _body(*refs):
    it = iter(refs)
    ent_h = next(it)
    ent_t = next(it)
    rel_f = next(it)
    rel_i = next(it)
    rtc = next(it)
    stw = next(it)
    time_tabs = [next(it) for _ in range(18)]
    heads = next(it)
    tails = next(it)
    rels = next(it)
    dates = next(it)
    datesd = next(it)
    yrf = next(it)
    mof = next(it)
    dyf = next(it)
    out = next(it)
    i_heads = next(it)
    i_tails = next(it)
    i_rels = next(it)
    i_dates = next(it)
    i_datesd = next(it)
    v_yr = next(it)
    v_mo = next(it)
    v_dy = next(it)
    b_h1s = next(it)
    b_t2s = next(it)
    b_h2s = next(it)
    b_t1s = next(it)
    b_rf = next(it)
    b_ri = next(it)
    b_tm = next(it)
    b_sw = next(it)
    b_time = [next(it) for _ in range(36)]
    v_score = next(it)
    sem = next(it)

    wid = lax.axis_index("s") * NC + lax.axis_index("c")

    def chunk_body(c, carry):
        base = wid * PER_W + c * CHUNK
        sl = pl.ds(base, CHUNK)
        pltpu.sync_copy(heads.at[sl], i_heads)
        pltpu.sync_copy(tails.at[sl], i_tails)
        pltpu.sync_copy(rels.at[sl], i_rels)
        pltpu.sync_copy(dates.at[sl], i_dates)
        pltpu.sync_copy(datesd.at[sl], i_datesd)
        pltpu.sync_copy(yrf.at[sl], v_yr)
        pltpu.sync_copy(mof.at[sl], v_mo)
        pltpu.sync_copy(dyf.at[sl], v_dy)

        cps = []

        def G(tbl, idxv, dst):
            cps.append(pltpu.async_copy(tbl.at[idxv], dst, sem))

        G(ent_h, i_heads, b_h1s)
        G(ent_t, i_heads, b_t2s)
        G(ent_h, i_tails, b_h2s)
        G(ent_t, i_tails, b_t1s)
        G(rel_f, i_rels, b_rf)
        G(rel_i, i_rels, b_ri)
        G(rtc, i_dates, b_tm)
        G(stw, i_datesd, b_sw)
        for srci, idxv in ((0, i_heads), (1, i_tails)):
            for j in range(18):
                G(time_tabs[j], idxv, b_time[srci * 18 + j])
        for cp in cps:
            cp.wait()

        def elem_body(el, carry2):
            eb = jnp.full((LANES,), el, jnp.int32)
            yr = plsc.load_gather(v_yr, [eb])
            mo = plsc.load_gather(v_mo, [eb])
            dy = plsc.load_gather(v_dy, [eb])

            acc = jnp.zeros((LANES,), jnp.float32)
            for v in range(S_DIM // LANES):
                cs = pl.ds(v * LANES, LANES)
                h1 = b_h1s[el, cs]
                t1 = b_t1s[el, cs]
                h2 = b_h2s[el, cs]
                t2 = b_t2s[el, cs]
                rf = b_rf[el, cs]
                ri = b_ri[el, cs]
                tm = b_tm[el, cs]
                r1 = rf + rf * tm
                r2 = ri + ri * tm
                acc = acc + h1 * r1 * t1 + h2 * r2 * t2

            for v in range(T_DIM // LANES):
                cs = pl.ds(v * LANES, LANES)
                cs96 = pl.ds(S_DIM + v * LANES, LANES)
                rf = b_rf[el, cs96]
                ri = b_ri[el, cs96]
                tm = b_tm[el, cs96]
                sw = b_sw[el, cs]

                def tte(srci, pfx):
                    tb = srci * 18 + pfx * 9
                    acc_t = None
                    for peri, tv in ((0, yr), (1, mo), (2, dy)):
                        fq = b_time[tb + 0 * 3 + peri][el, cs]
                        ph = b_time[tb + 1 * 3 + peri][el, cs]
                        am = b_time[tb + 2 * 3 + peri][el, cs]
                        term = am * _psin(fq * tv + ph)
                        acc_t = term if acc_t is None else acc_t + term
                    return acc_t + sw

                h1 = tte(0, 0)
                t1 = tte(1, 1)
                h2 = tte(1, 0)
                t2 = tte(0, 1)
                r1 = rf + rf * tm
                r2 = ri + ri * tm
                acc = acc + h1 * r1 * t1 + h2 * r2 * t2

            s = jnp.sum(acc) * 0.5
            lane = lax.iota(jnp.int32, 16)
            plsc.store_scatter(v_score, [eb], jnp.full((LANES,), s), mask=lane == 0)
            return carry2

        lax.fori_loop(0, CHUNK, elem_body, 0)
        pltpu.sync_copy(v_score, out.at[sl])
        return carry

    lax.fori_loop(0, NCHUNK, chunk_body, 0)


_SCRATCH = (
    [pltpu.VMEM((CHUNK,), jnp.int32)] * 5
    + [pltpu.VMEM((CHUNK,), jnp.float32)] * 3
    + [pltpu.VMEM((CHUNK, S_DIM), jnp.float32)] * 4
    + [pltpu.VMEM((CHUNK, R_DIM), jnp.float32)] * 3
    + [pltpu.VMEM((CHUNK, T_DIM), jnp.float32)]
    + [pltpu.VMEM((CHUNK, T_DIM), jnp.float32)] * 36
    + [pltpu.VMEM((CHUNK,), jnp.float32)]
    + [pltpu.SemaphoreType.DMA]
)

_sc_call = pl.kernel(
    _body,
    out_type=jax.ShapeDtypeStruct((B,), jnp.float32),
    mesh=plsc.VectorSubcoreMesh(core_axis_name="c", subcore_axis_name="s"),
    scratch_types=_SCRATCH,
    compiler_params=pltpu.CompilerParams(
        needs_layout_passes=False, use_tc_tiling_on_sc=False
    ),
)


def kernel(params, heads, rels, tails, years, months, days, date_ids):
    tables = [
        params["ent_embs_h"],
        params["ent_embs_t"],
        params["rel_embs_f"],
        params["rel_embs_i"],
        params["rtc"],
        params["stw"],
    ]
    tables += [
        params[f"{per}_{kind}_{pfx}"]
        for pfx in ("h", "t")
        for kind in ("freq", "phi", "amps")
        for per in ("y", "m", "d")
    ]
    args = tables + [
        heads.astype(jnp.int32),
        tails.astype(jnp.int32),
        rels.astype(jnp.int32),
        date_ids.astype(jnp.int32),
        (date_ids // CYCLE).astype(jnp.int32),
        years.astype(jnp.float32),
        months.astype(jnp.float32),
        days.astype(jnp.float32),
    ]
    return _sc_call(*args)
